# Initial kernel scaffold; baseline (speedup 1.0000x reference)
import jax
import jax.numpy as jnp
from jax.experimental import pallas as pl


def kernel(x, edge_index, L1_k0_W1, L1_k0_W2, L1_k0_b, L1_k1_W1, L1_k1_W2, L1_k1_b, L2_W1, L2_W2, L2_b):
    raise NotImplementedError("write your pallas kernel here")


# speedup vs baseline: 30.7542x; 30.7542x over previous
import functools

import jax
import jax.numpy as jnp
from jax import lax
from jax.experimental import pallas as pl
from jax.experimental.pallas import tpu as pltpu
from jax.experimental.pallas import tpu_sc as plsc

N, E, F, C, OUT = 10000, 320000, 128, 16, 7
NC, NS = 2, 16
NW = NC * NS
EW = E // NW
CH = 80
NCH = EW // CH
RB = 1000
ZTILES = N // RB

@functools.cache
def _mesh():
    return plsc.VectorSubcoreMesh(
        core_axis_name="c", subcore_axis_name="s", num_cores=NC,
        num_subcores=NS)


def _deg_body(src3, dst3, ones_hbm, zeros_hbm, dego_p, degi_p,
              srcv, dstv, ones_v, dego_s, degi_s):
    cid = lax.axis_index("c")
    sid = lax.axis_index("s")
    wid = sid * NC + cid

    @pl.when(sid < ZTILES)
    def _():
        sl = pl.ds(sid * RB, RB)
        pltpu.sync_copy(zeros_hbm.at[sl], dego_s.at[sl])
        pltpu.sync_copy(zeros_hbm.at[sl], degi_s.at[sl])

    pltpu.sync_copy(src3.at[wid], srcv)
    pltpu.sync_copy(dst3.at[wid], dstv)
    pltpu.sync_copy(ones_hbm, ones_v)
    plsc.subcore_barrier()

    def chunk(i, carry):
        pltpu.sync_copy(ones_v, dego_s.at[srcv.at[i]], add=True)
        pltpu.sync_copy(ones_v, degi_s.at[dstv.at[i]], add=True)
        return carry

    lax.fori_loop(0, NCH, chunk, 0)
    plsc.subcore_barrier()

    @pl.when(sid < ZTILES)
    def _():
        sl = pl.ds(sid * RB, RB)
        pltpu.sync_copy(dego_s.at[sl], dego_p.at[cid, sl])
        pltpu.sync_copy(degi_s.at[sl], degi_p.at[cid, sl])


@functools.cache
def _deg_call():
    return pl.kernel(
        _deg_body,
        out_type=(
            jax.ShapeDtypeStruct((NC, N, 8), jnp.float32),
            jax.ShapeDtypeStruct((NC, N, 8), jnp.float32),
        ),
        mesh=_mesh(),
        scratch_types=[
            pltpu.VMEM((NCH, CH), jnp.int32),
            pltpu.VMEM((NCH, CH), jnp.int32),
            pltpu.VMEM((CH, 8), jnp.float32),
            pltpu.VMEM_SHARED((N, 8), jnp.float32),
            pltpu.VMEM_SHARED((N, 8), jnp.float32),
        ],
        compiler_params=pltpu.CompilerParams(use_tc_tiling_on_sc=False),
    )


def _prop_body(W, table_hbm, src3, dst3, zeros_hbm, part_hbm,
               srcv, dstv, rows, acc, sem0, sem1):
    cid = lax.axis_index("c")
    sid = lax.axis_index("s")
    wid = sid * NC + cid
    sems = (sem0, sem1)

    @pl.when(sid < ZTILES)
    def _():
        sl = pl.ds(sid * RB, RB)
        pltpu.sync_copy(zeros_hbm.at[sl], acc.at[sl])

    pltpu.sync_copy(src3.at[wid], srcv)
    pltpu.sync_copy(dst3.at[wid], dstv)
    plsc.subcore_barrier()

    for b in range(2):
        pltpu.async_copy(table_hbm.at[srcv.at[b]], rows.at[b], sems[b])

    def step(g, carry):
        for b in range(2):
            i = 2 * g + b
            pltpu.make_async_copy(
                table_hbm.at[srcv.at[i]], rows.at[b], sems[b]).wait()
            pltpu.sync_copy(rows.at[b], acc.at[dstv.at[i]], add=True)
            nxt = i + 2

            @pl.when(nxt < NCH)
            def _():
                pltpu.async_copy(table_hbm.at[srcv.at[nxt]], rows.at[b],
                                 sems[b])
        return carry

    lax.fori_loop(0, NCH // 2, step, 0)
    if NCH % 2:
        bt = (NCH - 1) % 2
        pltpu.make_async_copy(
            table_hbm.at[srcv.at[NCH - 1]], rows.at[bt], sems[bt]).wait()
        pltpu.sync_copy(rows.at[bt], acc.at[dstv.at[NCH - 1]], add=True)

    plsc.subcore_barrier()

    @pl.when(sid < ZTILES)
    def _():
        sl = pl.ds(sid * RB, RB)
        pltpu.sync_copy(acc.at[sl], part_hbm.at[cid, sl])


@functools.cache
def _make_prop(W):
    return pl.kernel(
        functools.partial(_prop_body, W),
        out_type=jax.ShapeDtypeStruct((NC, N, W), jnp.float32),
        mesh=_mesh(),
        scratch_types=[
            pltpu.VMEM((NCH, CH), jnp.int32),
            pltpu.VMEM((NCH, CH), jnp.int32),
            pltpu.VMEM((2, CH, W), jnp.float32),
            pltpu.VMEM_SHARED((N, W), jnp.float32),
            pltpu.SemaphoreType.DMA,
            pltpu.SemaphoreType.DMA,
        ],
        compiler_params=pltpu.CompilerParams(use_tc_tiling_on_sc=False),
    )


def _elu(v):
    return jnp.where(v > 0, v, jnp.exp(jnp.minimum(v, 0.0)) - 1.0)


def _tcb_body(do_ref, di_ref, x_ref, w_ref, t_ref, a_ref, b_ref):
    a = lax.rsqrt(jnp.maximum(do_ref[0][:, :1] + do_ref[1][:, :1], 1.0))
    b = lax.rsqrt(jnp.maximum(di_ref[0][:, :1] + di_ref[1][:, :1], 1.0))
    xw = jnp.dot(x_ref[...], w_ref[...], preferred_element_type=jnp.float32)
    t_ref[...] = xw * a
    a_ref[...] = a
    b_ref[...] = b


def _tcd_body(p_ref, a_ref, b_ref, x_ref, w2_ref, bias_ref, w1p_ref,
              h_ref, t2_ref):
    axw = (p_ref[0] + p_ref[1]) * b_ref[...]
    xw2 = jnp.dot(x_ref[...], w2_ref[...], preferred_element_type=jnp.float32)
    s = _elu(axw + xw2 + bias_ref[...])
    h = _elu(0.5 * (s[:, :C] + s[:, C:]))
    h_ref[...] = h
    t2_ref[...] = jnp.dot(h, w1p_ref[...],
                          preferred_element_type=jnp.float32) * a_ref[...]


def _tcf_body(p_ref, b_ref, h_ref, w2p_ref, bp_ref, o_ref):
    z = (p_ref[0] + p_ref[1]) * b_ref[...]
    logits = z + jnp.dot(h_ref[...], w2p_ref[...],
                         preferred_element_type=jnp.float32) + bp_ref[...]
    m = jnp.max(logits, axis=1, keepdims=True)
    e = jnp.exp(logits - m)
    o_ref[...] = (e / jnp.sum(e, axis=1, keepdims=True))[:, :OUT]


def _row_spec(w):
    return pl.BlockSpec((RB, w), lambda i: (i, 0))


def _part_spec(w):
    return pl.BlockSpec((NC, RB, w), lambda i: (0, i, 0))


def _full_spec(r, w):
    return pl.BlockSpec((r, w), lambda i: (0, 0))


_tcB = pl.pallas_call(
    _tcb_body,
    grid=(N // RB,),
    in_specs=[_part_spec(8), _part_spec(8), _row_spec(F), _full_spec(F, 2 * C)],
    out_specs=[_row_spec(2 * C), _row_spec(1), _row_spec(1)],
    out_shape=[
        jax.ShapeDtypeStruct((N, 2 * C), jnp.float32),
        jax.ShapeDtypeStruct((N, 1), jnp.float32),
        jax.ShapeDtypeStruct((N, 1), jnp.float32),
    ],
)

_tcD = pl.pallas_call(
    _tcd_body,
    grid=(N // RB,),
    in_specs=[_part_spec(2 * C), _row_spec(1), _row_spec(1), _row_spec(F),
              _full_spec(F, 2 * C), _full_spec(1, 2 * C), _full_spec(C, 8)],
    out_specs=[_row_spec(C), _row_spec(8)],
    out_shape=[
        jax.ShapeDtypeStruct((N, C), jnp.float32),
        jax.ShapeDtypeStruct((N, 8), jnp.float32),
    ],
)

_tcF = pl.pallas_call(
    _tcf_body,
    grid=(N // RB,),
    in_specs=[_part_spec(8), _row_spec(1), _row_spec(C),
              _full_spec(C, 8), _full_spec(1, 8)],
    out_specs=_row_spec(OUT),
    out_shape=jax.ShapeDtypeStruct((N, OUT), jnp.float32),
)


def kernel(x, edge_index, L1_k0_W1, L1_k0_W2, L1_k0_b, L1_k1_W1, L1_k1_W2,
           L1_k1_b, L2_W1, L2_W2, L2_b):
    src3 = edge_index[0].reshape(NW, NCH, CH)
    dst3 = edge_index[1].reshape(NW, NCH, CH)
    ones_ch = jnp.ones((CH, 8), jnp.float32)
    zeros_n32 = jnp.zeros((N, 2 * C), jnp.float32)
    zeros_n8 = jnp.zeros((N, 8), jnp.float32)

    dego_p, degi_p = _deg_call()(src3, dst3, ones_ch, zeros_n8)

    wcat = jnp.concatenate([L1_k0_W1, L1_k1_W1], axis=1)
    table1, a_n1, b_n1 = _tcB(dego_p, degi_p, x, wcat)

    part1 = _make_prop(2 * C)(table1, src3, dst3, zeros_n32)

    w2cat = jnp.concatenate([L1_k0_W2, L1_k1_W2], axis=1)
    bias01 = jnp.concatenate([L1_k0_b, L1_k1_b]).reshape(1, 2 * C)
    w1p = jnp.pad(L2_W1, ((0, 0), (0, 1)))
    h, table2 = _tcD(part1, a_n1, b_n1, x, w2cat, bias01, w1p)

    part2 = _make_prop(8)(table2, src3, dst3, zeros_n8)

    w2p = jnp.pad(L2_W2, ((0, 0), (0, 1)))
    bp = jnp.concatenate(
        [L2_b, jnp.full((1,), -1e30, jnp.float32)]).reshape(1, 8)
    return _tcF(part2, b_n1, h, w2p, bp)


# speedup vs baseline: 42.9456x; 1.3964x over previous
import functools

import jax
import jax.numpy as jnp
from jax import lax
from jax.experimental import pallas as pl
from jax.experimental.pallas import tpu as pltpu
from jax.experimental.pallas import tpu_sc as plsc

N, E, F, C, OUT = 10000, 320000, 128, 16, 7
NC, NS = 2, 16
NW = NC * NS
EW = E // NW
CH = 1000
NCH = EW // CH
RB = 1000
ZTILES = N // RB

@functools.cache
def _mesh():
    return plsc.VectorSubcoreMesh(
        core_axis_name="c", subcore_axis_name="s", num_cores=NC,
        num_subcores=NS)


def _deg_body(src3, dst3, ones_hbm, zeros_hbm, dego_p, degi_p,
              srcv, dstv, ones_v, dego_s, degi_s):
    cid = lax.axis_index("c")
    sid = lax.axis_index("s")
    wid = sid * NC + cid

    @pl.when(sid < ZTILES)
    def _():
        sl = pl.ds(sid * RB, RB)
        pltpu.sync_copy(zeros_hbm.at[sl], dego_s.at[sl])
        pltpu.sync_copy(zeros_hbm.at[sl], degi_s.at[sl])

    pltpu.sync_copy(src3.at[wid], srcv)
    pltpu.sync_copy(dst3.at[wid], dstv)
    pltpu.sync_copy(ones_hbm, ones_v)
    plsc.subcore_barrier()

    def chunk(i, carry):
        pltpu.sync_copy(ones_v, dego_s.at[srcv.at[i]], add=True)
        pltpu.sync_copy(ones_v, degi_s.at[dstv.at[i]], add=True)
        return carry

    lax.fori_loop(0, NCH, chunk, 0)
    plsc.subcore_barrier()

    @pl.when(sid < ZTILES)
    def _():
        sl = pl.ds(sid * RB, RB)
        pltpu.sync_copy(dego_s.at[sl], dego_p.at[cid, sl])
        pltpu.sync_copy(degi_s.at[sl], degi_p.at[cid, sl])


@functools.cache
def _deg_call():
    return pl.kernel(
        _deg_body,
        out_type=(
            jax.ShapeDtypeStruct((NC, N, 8), jnp.float32),
            jax.ShapeDtypeStruct((NC, N, 8), jnp.float32),
        ),
        mesh=_mesh(),
        scratch_types=[
            pltpu.VMEM((NCH, CH), jnp.int32),
            pltpu.VMEM((NCH, CH), jnp.int32),
            pltpu.VMEM((CH, 8), jnp.float32),
            pltpu.VMEM_SHARED((N, 8), jnp.float32),
            pltpu.VMEM_SHARED((N, 8), jnp.float32),
        ],
        compiler_params=pltpu.CompilerParams(use_tc_tiling_on_sc=False),
    )


def _prop_body(W, table_hbm, src3, dst3, zeros_hbm, part_hbm,
               srcv, dstv, rows, acc, sem0, sem1):
    cid = lax.axis_index("c")
    sid = lax.axis_index("s")
    wid = sid * NC + cid
    sems = (sem0, sem1)

    @pl.when(sid < ZTILES)
    def _():
        sl = pl.ds(sid * RB, RB)
        pltpu.sync_copy(zeros_hbm.at[sl], acc.at[sl])

    pltpu.sync_copy(src3.at[wid], srcv)
    pltpu.sync_copy(dst3.at[wid], dstv)
    plsc.subcore_barrier()

    for b in range(2):
        pltpu.async_copy(table_hbm.at[srcv.at[b]], rows.at[b], sems[b])

    def step(g, carry):
        for b in range(2):
            i = 2 * g + b
            pltpu.make_async_copy(
                table_hbm.at[srcv.at[i]], rows.at[b], sems[b]).wait()
            pltpu.sync_copy(rows.at[b], acc.at[dstv.at[i]], add=True)
            nxt = i + 2

            @pl.when(nxt < NCH)
            def _():
                pltpu.async_copy(table_hbm.at[srcv.at[nxt]], rows.at[b],
                                 sems[b])
        return carry

    lax.fori_loop(0, NCH // 2, step, 0)
    if NCH % 2:
        bt = (NCH - 1) % 2
        pltpu.make_async_copy(
            table_hbm.at[srcv.at[NCH - 1]], rows.at[bt], sems[bt]).wait()
        pltpu.sync_copy(rows.at[bt], acc.at[dstv.at[NCH - 1]], add=True)

    plsc.subcore_barrier()

    @pl.when(sid < ZTILES)
    def _():
        sl = pl.ds(sid * RB, RB)
        pltpu.sync_copy(acc.at[sl], part_hbm.at[cid, sl])


@functools.cache
def _make_prop(W):
    return pl.kernel(
        functools.partial(_prop_body, W),
        out_type=jax.ShapeDtypeStruct((NC, N, W), jnp.float32),
        mesh=_mesh(),
        scratch_types=[
            pltpu.VMEM((NCH, CH), jnp.int32),
            pltpu.VMEM((NCH, CH), jnp.int32),
            pltpu.VMEM((2, CH, W), jnp.float32),
            pltpu.VMEM_SHARED((N, W), jnp.float32),
            pltpu.SemaphoreType.DMA,
            pltpu.SemaphoreType.DMA,
        ],
        compiler_params=pltpu.CompilerParams(use_tc_tiling_on_sc=False),
    )


def _elu(v):
    return jnp.where(v > 0, v, jnp.exp(jnp.minimum(v, 0.0)) - 1.0)


def _tcb_body(do_ref, di_ref, x_ref, w_ref, t_ref, a_ref, b_ref):
    a = lax.rsqrt(jnp.maximum(do_ref[0][:, :1] + do_ref[1][:, :1], 1.0))
    b = lax.rsqrt(jnp.maximum(di_ref[0][:, :1] + di_ref[1][:, :1], 1.0))
    xw = jnp.dot(x_ref[...], w_ref[...], preferred_element_type=jnp.float32)
    t_ref[...] = xw * a
    a_ref[...] = a
    b_ref[...] = b


def _tcd_body(p_ref, a_ref, b_ref, x_ref, w2_ref, bias_ref, w1p_ref,
              h_ref, t2_ref):
    axw = (p_ref[0] + p_ref[1]) * b_ref[...]
    xw2 = jnp.dot(x_ref[...], w2_ref[...], preferred_element_type=jnp.float32)
    s = _elu(axw + xw2 + bias_ref[...])
    h = _elu(0.5 * (s[:, :C] + s[:, C:]))
    h_ref[...] = h
    t2_ref[...] = jnp.dot(h, w1p_ref[...],
                          preferred_element_type=jnp.float32) * a_ref[...]


def _tcf_body(p_ref, b_ref, h_ref, w2p_ref, bp_ref, o_ref):
    z = (p_ref[0] + p_ref[1]) * b_ref[...]
    logits = z + jnp.dot(h_ref[...], w2p_ref[...],
                         preferred_element_type=jnp.float32) + bp_ref[...]
    m = jnp.max(logits, axis=1, keepdims=True)
    e = jnp.exp(logits - m)
    o_ref[...] = (e / jnp.sum(e, axis=1, keepdims=True))[:, :OUT]


def _row_spec(w):
    return pl.BlockSpec((RB, w), lambda i: (i, 0))


def _part_spec(w):
    return pl.BlockSpec((NC, RB, w), lambda i: (0, i, 0))


def _full_spec(r, w):
    return pl.BlockSpec((r, w), lambda i: (0, 0))


_tcB = pl.pallas_call(
    _tcb_body,
    grid=(N // RB,),
    in_specs=[_part_spec(8), _part_spec(8), _row_spec(F), _full_spec(F, 2 * C)],
    out_specs=[_row_spec(2 * C), _row_spec(1), _row_spec(1)],
    out_shape=[
        jax.ShapeDtypeStruct((N, 2 * C), jnp.float32),
        jax.ShapeDtypeStruct((N, 1), jnp.float32),
        jax.ShapeDtypeStruct((N, 1), jnp.float32),
    ],
)

_tcD = pl.pallas_call(
    _tcd_body,
    grid=(N // RB,),
    in_specs=[_part_spec(2 * C), _row_spec(1), _row_spec(1), _row_spec(F),
              _full_spec(F, 2 * C), _full_spec(1, 2 * C), _full_spec(C, 8)],
    out_specs=[_row_spec(C), _row_spec(8)],
    out_shape=[
        jax.ShapeDtypeStruct((N, C), jnp.float32),
        jax.ShapeDtypeStruct((N, 8), jnp.float32),
    ],
)

_tcF = pl.pallas_call(
    _tcf_body,
    grid=(N // RB,),
    in_specs=[_part_spec(8), _row_spec(1), _row_spec(C),
              _full_spec(C, 8), _full_spec(1, 8)],
    out_specs=_row_spec(OUT),
    out_shape=jax.ShapeDtypeStruct((N, OUT), jnp.float32),
)


def kernel(x, edge_index, L1_k0_W1, L1_k0_W2, L1_k0_b, L1_k1_W1, L1_k1_W2,
           L1_k1_b, L2_W1, L2_W2, L2_b):
    src3 = edge_index[0].reshape(NW, NCH, CH)
    dst3 = edge_index[1].reshape(NW, NCH, CH)
    ones_ch = jnp.ones((CH, 8), jnp.float32)
    zeros_n32 = jnp.zeros((N, 2 * C), jnp.float32)
    zeros_n8 = jnp.zeros((N, 8), jnp.float32)

    dego_p, degi_p = _deg_call()(src3, dst3, ones_ch, zeros_n8)

    wcat = jnp.concatenate([L1_k0_W1, L1_k1_W1], axis=1)
    table1, a_n1, b_n1 = _tcB(dego_p, degi_p, x, wcat)

    part1 = _make_prop(2 * C)(table1, src3, dst3, zeros_n32)

    w2cat = jnp.concatenate([L1_k0_W2, L1_k1_W2], axis=1)
    bias01 = jnp.concatenate([L1_k0_b, L1_k1_b]).reshape(1, 2 * C)
    w1p = jnp.pad(L2_W1, ((0, 0), (0, 1)))
    h, table2 = _tcD(part1, a_n1, b_n1, x, w2cat, bias01, w1p)

    part2 = _make_prop(8)(table2, src3, dst3, zeros_n8)

    w2p = jnp.pad(L2_W2, ((0, 0), (0, 1)))
    bp = jnp.concatenate(
        [L2_b, jnp.full((1,), -1e30, jnp.float32)]).reshape(1, 8)
    return _tcF(part2, b_n1, h, w2p, bp)


# speedup vs baseline: 42.9690x; 1.0005x over previous
import functools

import jax
import jax.numpy as jnp
from jax import lax
from jax.experimental import pallas as pl
from jax.experimental.pallas import tpu as pltpu
from jax.experimental.pallas import tpu_sc as plsc

N, E, F, C, OUT = 10000, 320000, 128, 16, 7
NC, NS = 2, 16
NW = NC * NS
EW = E // NW
CH = 1000
NCH = EW // CH
RB = 1000
ZTILES = N // RB

@functools.cache
def _mesh():
    return plsc.VectorSubcoreMesh(
        core_axis_name="c", subcore_axis_name="s", num_cores=NC,
        num_subcores=NS)


def _deg_body(src3, dst3, ones_hbm, zeros_hbm, dego_p, degi_p,
              srcv, dstv, ones_v, dego_s, degi_s, sem):
    cid = lax.axis_index("c")
    sid = lax.axis_index("s")
    wid = sid * NC + cid

    @pl.when(sid < ZTILES)
    def _():
        sl = pl.ds(sid * RB, RB)
        pltpu.sync_copy(zeros_hbm.at[sl], dego_s.at[sl])
        pltpu.sync_copy(zeros_hbm.at[sl], degi_s.at[sl])

    pltpu.sync_copy(src3.at[wid], srcv)
    pltpu.sync_copy(dst3.at[wid], dstv)
    pltpu.sync_copy(ones_hbm, ones_v)
    plsc.subcore_barrier()

    def chunk(i, carry):
        pltpu.async_copy(ones_v, dego_s.at[srcv.at[i]], sem, add=True)
        pltpu.async_copy(ones_v, degi_s.at[dstv.at[i]], sem, add=True)
        return carry

    lax.fori_loop(0, NCH, chunk, 0)

    def drain(i, carry):
        pltpu.make_async_copy(ones_v, dego_s.at[srcv.at[0]], sem).wait()
        pltpu.make_async_copy(ones_v, degi_s.at[dstv.at[0]], sem).wait()
        return carry

    lax.fori_loop(0, NCH, drain, 0)
    plsc.subcore_barrier()

    @pl.when(sid < ZTILES)
    def _():
        sl = pl.ds(sid * RB, RB)
        pltpu.sync_copy(dego_s.at[sl], dego_p.at[cid, sl])
        pltpu.sync_copy(degi_s.at[sl], degi_p.at[cid, sl])


@functools.cache
def _deg_call():
    return pl.kernel(
        _deg_body,
        out_type=(
            jax.ShapeDtypeStruct((NC, N, 8), jnp.float32),
            jax.ShapeDtypeStruct((NC, N, 8), jnp.float32),
        ),
        mesh=_mesh(),
        scratch_types=[
            pltpu.VMEM((NCH, CH), jnp.int32),
            pltpu.VMEM((NCH, CH), jnp.int32),
            pltpu.VMEM((CH, 8), jnp.float32),
            pltpu.VMEM_SHARED((N, 8), jnp.float32),
            pltpu.VMEM_SHARED((N, 8), jnp.float32),
            pltpu.SemaphoreType.DMA,
        ],
        compiler_params=pltpu.CompilerParams(use_tc_tiling_on_sc=False),
    )


def _prop_body(W, table_hbm, src3, dst3, zeros_hbm, part_hbm,
               srcv, dstv, rows, acc, gsem0, gsem1, ssem0, ssem1):
    cid = lax.axis_index("c")
    sid = lax.axis_index("s")
    wid = sid * NC + cid
    gsems = (gsem0, gsem1)
    ssems = (ssem0, ssem1)

    @pl.when(sid < ZTILES)
    def _():
        sl = pl.ds(sid * RB, RB)
        pltpu.sync_copy(zeros_hbm.at[sl], acc.at[sl])

    pltpu.sync_copy(src3.at[wid], srcv)
    pltpu.sync_copy(dst3.at[wid], dstv)
    plsc.subcore_barrier()

    for b in range(2):
        pltpu.async_copy(table_hbm.at[srcv.at[b]], rows.at[b], gsems[b])

    def step(g, carry):
        for b in range(2):
            i = 2 * g + b
            pltpu.make_async_copy(
                table_hbm.at[srcv.at[i]], rows.at[b], gsems[b]).wait()
            pltpu.async_copy(rows.at[b], acc.at[dstv.at[i]], ssems[b],
                             add=True)
            nxt = i + 2

            @pl.when(nxt < NCH)
            def _():
                pltpu.make_async_copy(
                    rows.at[b], acc.at[dstv.at[i]], ssems[b]).wait()
                pltpu.async_copy(table_hbm.at[srcv.at[nxt]], rows.at[b],
                                 gsems[b])
        return carry

    lax.fori_loop(0, NCH // 2, step, 0)
    for b in range(2):
        pltpu.make_async_copy(
            rows.at[b], acc.at[dstv.at[NCH - 2 + b]], ssems[b]).wait()

    plsc.subcore_barrier()

    @pl.when(sid < ZTILES)
    def _():
        sl = pl.ds(sid * RB, RB)
        pltpu.sync_copy(acc.at[sl], part_hbm.at[cid, sl])


@functools.cache
def _make_prop(W):
    return pl.kernel(
        functools.partial(_prop_body, W),
        out_type=jax.ShapeDtypeStruct((NC, N, W), jnp.float32),
        mesh=_mesh(),
        scratch_types=[
            pltpu.VMEM((NCH, CH), jnp.int32),
            pltpu.VMEM((NCH, CH), jnp.int32),
            pltpu.VMEM((2, CH, W), jnp.float32),
            pltpu.VMEM_SHARED((N, W), jnp.float32),
            pltpu.SemaphoreType.DMA,
            pltpu.SemaphoreType.DMA,
            pltpu.SemaphoreType.DMA,
            pltpu.SemaphoreType.DMA,
        ],
        compiler_params=pltpu.CompilerParams(use_tc_tiling_on_sc=False),
    )


def _elu(v):
    return jnp.where(v > 0, v, jnp.exp(jnp.minimum(v, 0.0)) - 1.0)


def _tcb_body(do_ref, di_ref, x_ref, w_ref, t_ref, a_ref, b_ref):
    a = lax.rsqrt(jnp.maximum(do_ref[0][:, :1] + do_ref[1][:, :1], 1.0))
    b = lax.rsqrt(jnp.maximum(di_ref[0][:, :1] + di_ref[1][:, :1], 1.0))
    xw = jnp.dot(x_ref[...], w_ref[...], preferred_element_type=jnp.float32)
    t_ref[...] = xw * a
    a_ref[...] = a
    b_ref[...] = b


def _tcd_body(p_ref, a_ref, b_ref, x_ref, w2_ref, bias_ref, w1p_ref,
              h_ref, t2_ref):
    axw = (p_ref[0] + p_ref[1]) * b_ref[...]
    xw2 = jnp.dot(x_ref[...], w2_ref[...], preferred_element_type=jnp.float32)
    s = _elu(axw + xw2 + bias_ref[...])
    h = _elu(0.5 * (s[:, :C] + s[:, C:]))
    h_ref[...] = h
    t2_ref[...] = jnp.dot(h, w1p_ref[...],
                          preferred_element_type=jnp.float32) * a_ref[...]


def _tcf_body(p_ref, b_ref, h_ref, w2p_ref, bp_ref, o_ref):
    z = (p_ref[0] + p_ref[1]) * b_ref[...]
    logits = z + jnp.dot(h_ref[...], w2p_ref[...],
                         preferred_element_type=jnp.float32) + bp_ref[...]
    m = jnp.max(logits, axis=1, keepdims=True)
    e = jnp.exp(logits - m)
    o_ref[...] = (e / jnp.sum(e, axis=1, keepdims=True))[:, :OUT]


def _row_spec(w):
    return pl.BlockSpec((RB, w), lambda i: (i, 0))


def _part_spec(w):
    return pl.BlockSpec((NC, RB, w), lambda i: (0, i, 0))


def _full_spec(r, w):
    return pl.BlockSpec((r, w), lambda i: (0, 0))


_tcB = pl.pallas_call(
    _tcb_body,
    grid=(N // RB,),
    in_specs=[_part_spec(8), _part_spec(8), _row_spec(F), _full_spec(F, 2 * C)],
    out_specs=[_row_spec(2 * C), _row_spec(1), _row_spec(1)],
    out_shape=[
        jax.ShapeDtypeStruct((N, 2 * C), jnp.float32),
        jax.ShapeDtypeStruct((N, 1), jnp.float32),
        jax.ShapeDtypeStruct((N, 1), jnp.float32),
    ],
)

_tcD = pl.pallas_call(
    _tcd_body,
    grid=(N // RB,),
    in_specs=[_part_spec(2 * C), _row_spec(1), _row_spec(1), _row_spec(F),
              _full_spec(F, 2 * C), _full_spec(1, 2 * C), _full_spec(C, 8)],
    out_specs=[_row_spec(C), _row_spec(8)],
    out_shape=[
        jax.ShapeDtypeStruct((N, C), jnp.float32),
        jax.ShapeDtypeStruct((N, 8), jnp.float32),
    ],
)

_tcF = pl.pallas_call(
    _tcf_body,
    grid=(N // RB,),
    in_specs=[_part_spec(8), _row_spec(1), _row_spec(C),
              _full_spec(C, 8), _full_spec(1, 8)],
    out_specs=_row_spec(OUT),
    out_shape=jax.ShapeDtypeStruct((N, OUT), jnp.float32),
)


def kernel(x, edge_index, L1_k0_W1, L1_k0_W2, L1_k0_b, L1_k1_W1, L1_k1_W2,
           L1_k1_b, L2_W1, L2_W2, L2_b):
    src3 = edge_index[0].reshape(NW, NCH, CH)
    dst3 = edge_index[1].reshape(NW, NCH, CH)
    ones_ch = jnp.ones((CH, 8), jnp.float32)
    zeros_n32 = jnp.zeros((N, 2 * C), jnp.float32)
    zeros_n8 = jnp.zeros((N, 8), jnp.float32)

    dego_p, degi_p = _deg_call()(src3, dst3, ones_ch, zeros_n8)

    wcat = jnp.concatenate([L1_k0_W1, L1_k1_W1], axis=1)
    table1, a_n1, b_n1 = _tcB(dego_p, degi_p, x, wcat)

    part1 = _make_prop(2 * C)(table1, src3, dst3, zeros_n32)

    w2cat = jnp.concatenate([L1_k0_W2, L1_k1_W2], axis=1)
    bias01 = jnp.concatenate([L1_k0_b, L1_k1_b]).reshape(1, 2 * C)
    w1p = jnp.pad(L2_W1, ((0, 0), (0, 1)))
    h, table2 = _tcD(part1, a_n1, b_n1, x, w2cat, bias01, w1p)

    part2 = _make_prop(8)(table2, src3, dst3, zeros_n8)

    w2p = jnp.pad(L2_W2, ((0, 0), (0, 1)))
    bp = jnp.concatenate(
        [L2_b, jnp.full((1,), -1e30, jnp.float32)]).reshape(1, 8)
    return _tcF(part2, b_n1, h, w2p, bp)


# speedup vs baseline: 47.3864x; 1.1028x over previous
import functools

import jax
import jax.numpy as jnp
from jax import lax
from jax.experimental import pallas as pl
from jax.experimental.pallas import tpu as pltpu
from jax.experimental.pallas import tpu_sc as plsc

N, E, F, C, OUT = 10000, 320000, 128, 16, 7
NC, NS = 2, 16
NW = NC * NS
EW = E // NW
CH = 1000
NCH = EW // CH
RB = 1000
ZTILES = N // RB

@functools.cache
def _mesh():
    return plsc.VectorSubcoreMesh(
        core_axis_name="c", subcore_axis_name="s", num_cores=NC,
        num_subcores=NS)


def _deg_body(src3, dst3, ones_hbm, zeros_hbm, dego_p, degi_p,
              srcv, dstv, ones_v, dego_s, degi_s, stage_v, outv, sem):
    cid = lax.axis_index("c")
    sid = lax.axis_index("s")
    wid = sid * NC + cid

    @pl.when(sid < ZTILES)
    def _():
        sl = pl.ds(sid * RB, RB)
        pltpu.sync_copy(zeros_hbm.at[sl], dego_s.at[sl])
        pltpu.sync_copy(zeros_hbm.at[sl], degi_s.at[sl])

    pltpu.sync_copy(src3.at[wid], srcv)
    pltpu.sync_copy(dst3.at[wid], dstv)
    pltpu.sync_copy(ones_hbm, ones_v)
    plsc.subcore_barrier()

    def chunk(i, carry):
        pltpu.async_copy(ones_v, dego_s.at[srcv.at[i]], sem, add=True)
        pltpu.async_copy(ones_v, degi_s.at[dstv.at[i]], sem, add=True)
        return carry

    lax.fori_loop(0, NCH, chunk, 0)

    def drain(i, carry):
        pltpu.make_async_copy(ones_v, dego_s.at[srcv.at[0]], sem).wait()
        pltpu.make_async_copy(ones_v, degi_s.at[dstv.at[0]], sem).wait()
        return carry

    lax.fori_loop(0, NCH, drain, 0)
    plsc.subcore_barrier()

    @pl.when(sid < ZTILES)
    def _():
        sl = pl.ds(sid * RB, RB)
        for acc_s, out_p in ((dego_s, dego_p), (degi_s, degi_p)):
            pltpu.sync_copy(acc_s.at[sl], stage_v.at[pl.ds(0, RB)])
            def ext(j, carry):
                rows = j * 16 + jnp.arange(16, dtype=jnp.int32)
                cols = jnp.zeros((16,), jnp.int32)
                vals = plsc.load_gather(stage_v, [rows, cols])
                outv[pl.ds(j * 16, 16)] = vals
                return carry
            lax.fori_loop(0, (RB + 15) // 16, ext, 0)
            pltpu.sync_copy(outv.at[pl.ds(0, RB)], out_p.at[sid, cid])


@functools.cache
def _deg_call():
    return pl.kernel(
        _deg_body,
        out_type=(
            jax.ShapeDtypeStruct((N // RB, NC, RB), jnp.float32),
            jax.ShapeDtypeStruct((N // RB, NC, RB), jnp.float32),
        ),
        mesh=_mesh(),
        scratch_types=[
            pltpu.VMEM((NCH, CH), jnp.int32),
            pltpu.VMEM((NCH, CH), jnp.int32),
            pltpu.VMEM((CH, 8), jnp.float32),
            pltpu.VMEM_SHARED((N, 8), jnp.float32),
            pltpu.VMEM_SHARED((N, 8), jnp.float32),
            pltpu.VMEM((RB + 8, 8), jnp.float32),
            pltpu.VMEM((RB + 8,), jnp.float32),
            pltpu.SemaphoreType.DMA,
        ],
        compiler_params=pltpu.CompilerParams(
            use_tc_tiling_on_sc=False, needs_layout_passes=False),
    )


def _prop_body(W, table_hbm, src3, dst3, zeros_hbm, part_hbm,
               srcv, dstv, rows, acc, gsem0, gsem1, ssem0, ssem1):
    cid = lax.axis_index("c")
    sid = lax.axis_index("s")
    wid = sid * NC + cid
    gsems = (gsem0, gsem1)
    ssems = (ssem0, ssem1)

    @pl.when(sid < ZTILES)
    def _():
        sl = pl.ds(sid * RB, RB)
        pltpu.sync_copy(zeros_hbm.at[sl], acc.at[sl])

    pltpu.sync_copy(src3.at[wid], srcv)
    pltpu.sync_copy(dst3.at[wid], dstv)
    plsc.subcore_barrier()

    for b in range(2):
        pltpu.async_copy(table_hbm.at[srcv.at[b]], rows.at[b], gsems[b])

    def step(g, carry):
        for b in range(2):
            i = 2 * g + b
            pltpu.make_async_copy(
                table_hbm.at[srcv.at[i]], rows.at[b], gsems[b]).wait()
            pltpu.async_copy(rows.at[b], acc.at[dstv.at[i]], ssems[b],
                             add=True)
            nxt = i + 2

            @pl.when(nxt < NCH)
            def _():
                pltpu.make_async_copy(
                    rows.at[b], acc.at[dstv.at[i]], ssems[b]).wait()
                pltpu.async_copy(table_hbm.at[srcv.at[nxt]], rows.at[b],
                                 gsems[b])
        return carry

    lax.fori_loop(0, NCH // 2, step, 0)
    for b in range(2):
        pltpu.make_async_copy(
            rows.at[b], acc.at[dstv.at[NCH - 2 + b]], ssems[b]).wait()

    plsc.subcore_barrier()

    @pl.when(sid < ZTILES)
    def _():
        sl = pl.ds(sid * RB, RB)
        pltpu.sync_copy(acc.at[sl], part_hbm.at[cid, sl])


@functools.cache
def _make_prop(W):
    return pl.kernel(
        functools.partial(_prop_body, W),
        out_type=jax.ShapeDtypeStruct((NC, N, W), jnp.float32),
        mesh=_mesh(),
        scratch_types=[
            pltpu.VMEM((NCH, CH), jnp.int32),
            pltpu.VMEM((NCH, CH), jnp.int32),
            pltpu.VMEM((2, CH, W), jnp.float32),
            pltpu.VMEM_SHARED((N, W), jnp.float32),
            pltpu.SemaphoreType.DMA,
            pltpu.SemaphoreType.DMA,
            pltpu.SemaphoreType.DMA,
            pltpu.SemaphoreType.DMA,
        ],
        compiler_params=pltpu.CompilerParams(use_tc_tiling_on_sc=False),
    )


def _elu(v):
    return jnp.where(v > 0, v, jnp.exp(jnp.minimum(v, 0.0)) - 1.0)


def _tcb_body(do_ref, di_ref, x_ref, w_ref, t_ref, a_ref, b_ref):
    a = lax.rsqrt(jnp.maximum(do_ref[0, 0] + do_ref[0, 1], 1.0)).reshape(RB, 1)
    b = lax.rsqrt(jnp.maximum(di_ref[0, 0] + di_ref[0, 1], 1.0)).reshape(RB, 1)
    xw = jnp.dot(x_ref[...], w_ref[...], preferred_element_type=jnp.float32)
    t_ref[...] = xw * a
    a_ref[...] = a
    b_ref[...] = b


def _tcd_body(p_ref, a_ref, b_ref, x_ref, w2_ref, bias_ref, w1p_ref,
              h_ref, t2_ref):
    axw = (p_ref[0] + p_ref[1]) * b_ref[...]
    xw2 = jnp.dot(x_ref[...], w2_ref[...], preferred_element_type=jnp.float32)
    s = _elu(axw + xw2 + bias_ref[...])
    h = _elu(0.5 * (s[:, :C] + s[:, C:]))
    h_ref[...] = h
    t2_ref[...] = jnp.dot(h, w1p_ref[...],
                          preferred_element_type=jnp.float32) * a_ref[...]


def _tcf_body(p_ref, b_ref, h_ref, w2p_ref, bp_ref, o_ref):
    z = (p_ref[0] + p_ref[1]) * b_ref[...]
    logits = z + jnp.dot(h_ref[...], w2p_ref[...],
                         preferred_element_type=jnp.float32) + bp_ref[...]
    m = jnp.max(logits, axis=1, keepdims=True)
    e = jnp.exp(logits - m)
    o_ref[...] = (e / jnp.sum(e, axis=1, keepdims=True))[:, :OUT]


def _row_spec(w):
    return pl.BlockSpec((RB, w), lambda i: (i, 0))


def _part_spec(w):
    return pl.BlockSpec((NC, RB, w), lambda i: (0, i, 0))


def _full_spec(r, w):
    return pl.BlockSpec((r, w), lambda i: (0, 0))


_tcB = pl.pallas_call(
    _tcb_body,
    grid=(N // RB,),
    in_specs=[pl.BlockSpec((1, NC, RB), lambda i: (i, 0, 0)),
              pl.BlockSpec((1, NC, RB), lambda i: (i, 0, 0)),
              _row_spec(F), _full_spec(F, 2 * C)],
    out_specs=[_row_spec(2 * C), _row_spec(1), _row_spec(1)],
    out_shape=[
        jax.ShapeDtypeStruct((N, 2 * C), jnp.float32),
        jax.ShapeDtypeStruct((N, 1), jnp.float32),
        jax.ShapeDtypeStruct((N, 1), jnp.float32),
    ],
)

_tcD = pl.pallas_call(
    _tcd_body,
    grid=(N // RB,),
    in_specs=[_part_spec(2 * C), _row_spec(1), _row_spec(1), _row_spec(F),
              _full_spec(F, 2 * C), _full_spec(1, 2 * C), _full_spec(C, 8)],
    out_specs=[_row_spec(C), _row_spec(8)],
    out_shape=[
        jax.ShapeDtypeStruct((N, C), jnp.float32),
        jax.ShapeDtypeStruct((N, 8), jnp.float32),
    ],
)

_tcF = pl.pallas_call(
    _tcf_body,
    grid=(N // RB,),
    in_specs=[_part_spec(8), _row_spec(1), _row_spec(C),
              _full_spec(C, 8), _full_spec(1, 8)],
    out_specs=_row_spec(OUT),
    out_shape=jax.ShapeDtypeStruct((N, OUT), jnp.float32),
)


def kernel(x, edge_index, L1_k0_W1, L1_k0_W2, L1_k0_b, L1_k1_W1, L1_k1_W2,
           L1_k1_b, L2_W1, L2_W2, L2_b):
    src3 = edge_index[0].reshape(NW, NCH, CH)
    dst3 = edge_index[1].reshape(NW, NCH, CH)
    ones_ch = jnp.ones((CH, 8), jnp.float32)
    zeros_n32 = jnp.zeros((N, 2 * C), jnp.float32)
    zeros_n8 = jnp.zeros((N, 8), jnp.float32)

    dego_p, degi_p = _deg_call()(src3, dst3, ones_ch, zeros_n8)

    wcat = jnp.concatenate([L1_k0_W1, L1_k1_W1], axis=1)
    table1, a_n1, b_n1 = _tcB(dego_p, degi_p, x, wcat)

    part1 = _make_prop(2 * C)(table1, src3, dst3, zeros_n32)

    w2cat = jnp.concatenate([L1_k0_W2, L1_k1_W2], axis=1)
    bias01 = jnp.concatenate([L1_k0_b, L1_k1_b]).reshape(1, 2 * C)
    w1p = jnp.pad(L2_W1, ((0, 0), (0, 1)))
    h, table2 = _tcD(part1, a_n1, b_n1, x, w2cat, bias01, w1p)

    part2 = _make_prop(8)(table2, src3, dst3, zeros_n8)

    w2p = jnp.pad(L2_W2, ((0, 0), (0, 1)))
    bp = jnp.concatenate(
        [L2_b, jnp.full((1,), -1e30, jnp.float32)]).reshape(1, 8)
    return _tcF(part2, b_n1, h, w2p, bp)


# speedup vs baseline: 48.4964x; 1.0234x over previous
import functools

import jax
import jax.numpy as jnp
from jax import lax
from jax.experimental import pallas as pl
from jax.experimental.pallas import tpu as pltpu
from jax.experimental.pallas import tpu_sc as plsc

N, E, F, C, OUT = 10000, 320000, 128, 16, 7
NC, NS = 2, 16
NW = NC * NS
EW = E // NW
CH = 1000
NCH = EW // CH
RB = 1000
ZTILES = N // RB

@functools.cache
def _mesh():
    return plsc.VectorSubcoreMesh(
        core_axis_name="c", subcore_axis_name="s", num_cores=NC,
        num_subcores=NS)


def _deg_body(src3, dst3, ones_hbm, zeros_hbm, dego_p, degi_p,
              srcv, dstv, ones_v, dego_s, degi_s, stage_v, outv, sem):
    cid = lax.axis_index("c")
    sid = lax.axis_index("s")
    wid = sid * NC + cid

    @pl.when(sid < ZTILES)
    def _():
        sl = pl.ds(sid * RB, RB)
        pltpu.sync_copy(zeros_hbm.at[sl], dego_s.at[sl])
        pltpu.sync_copy(zeros_hbm.at[sl], degi_s.at[sl])

    pltpu.sync_copy(src3.at[wid], srcv)
    pltpu.sync_copy(dst3.at[wid], dstv)
    pltpu.sync_copy(ones_hbm, ones_v)
    plsc.subcore_barrier()

    def chunk(i, carry):
        pltpu.async_copy(ones_v, dego_s.at[srcv.at[i]], sem, add=True)
        pltpu.async_copy(ones_v, degi_s.at[dstv.at[i]], sem, add=True)
        return carry

    lax.fori_loop(0, NCH, chunk, 0)

    def drain(i, carry):
        pltpu.make_async_copy(ones_v, dego_s.at[srcv.at[0]], sem).wait()
        pltpu.make_async_copy(ones_v, degi_s.at[dstv.at[0]], sem).wait()
        return carry

    lax.fori_loop(0, NCH, drain, 0)
    plsc.subcore_barrier()

    @pl.when(sid < ZTILES)
    def _():
        sl = pl.ds(sid * RB, RB)
        for acc_s, out_p in ((dego_s, dego_p), (degi_s, degi_p)):
            pltpu.sync_copy(acc_s.at[sl], stage_v.at[pl.ds(0, RB)])
            def ext(j, carry):
                rows = j * 16 + jnp.arange(16, dtype=jnp.int32)
                cols = jnp.zeros((16,), jnp.int32)
                vals = plsc.load_gather(stage_v, [rows, cols])
                outv[pl.ds(j * 16, 16)] = vals
                return carry
            lax.fori_loop(0, (RB + 15) // 16, ext, 0)
            pltpu.sync_copy(outv.at[pl.ds(0, RB)], out_p.at[sid, cid])


@functools.cache
def _deg_call():
    return pl.kernel(
        _deg_body,
        out_type=(
            jax.ShapeDtypeStruct((N // RB, NC, RB), jnp.float32),
            jax.ShapeDtypeStruct((N // RB, NC, RB), jnp.float32),
        ),
        mesh=_mesh(),
        scratch_types=[
            pltpu.VMEM((NCH, CH), jnp.int32),
            pltpu.VMEM((NCH, CH), jnp.int32),
            pltpu.VMEM((CH, 8), jnp.float32),
            pltpu.VMEM_SHARED((N, 8), jnp.float32),
            pltpu.VMEM_SHARED((N, 8), jnp.float32),
            pltpu.VMEM((RB + 8, 8), jnp.float32),
            pltpu.VMEM((RB + 8,), jnp.float32),
            pltpu.SemaphoreType.DMA,
        ],
        compiler_params=pltpu.CompilerParams(
            use_tc_tiling_on_sc=False, needs_layout_passes=False),
    )


def _prop_body(W, table_hbm, src3, dst3, zeros_hbm, part_hbm,
               srcv, dstv, rows, acc, gsem0, gsem1, ssem0, ssem1):
    cid = lax.axis_index("c")
    sid = lax.axis_index("s")
    wid = sid * NC + cid
    gsems = (gsem0, gsem1)
    ssems = (ssem0, ssem1)

    @pl.when(sid < ZTILES)
    def _():
        sl = pl.ds(sid * RB, RB)
        pltpu.sync_copy(zeros_hbm.at[sl], acc.at[sl])

    pltpu.sync_copy(src3.at[wid], srcv)
    pltpu.sync_copy(dst3.at[wid], dstv)
    plsc.subcore_barrier()

    for b in range(2):
        pltpu.async_copy(table_hbm.at[srcv.at[b]], rows.at[b], gsems[b])

    def step(g, carry):
        for b in range(2):
            i = 2 * g + b
            pltpu.make_async_copy(
                table_hbm.at[srcv.at[i]], rows.at[b], gsems[b]).wait()
            pltpu.async_copy(rows.at[b], acc.at[dstv.at[i]], ssems[b],
                             add=True)
            nxt = i + 2

            @pl.when(nxt < NCH)
            def _():
                pltpu.make_async_copy(
                    rows.at[b], acc.at[dstv.at[i]], ssems[b]).wait()
                pltpu.async_copy(table_hbm.at[srcv.at[nxt]], rows.at[b],
                                 gsems[b])
        return carry

    lax.fori_loop(0, NCH // 2, step, 0)
    for b in range(2):
        pltpu.make_async_copy(
            rows.at[b], acc.at[dstv.at[NCH - 2 + b]], ssems[b]).wait()

    plsc.subcore_barrier()

    @pl.when(sid < ZTILES)
    def _():
        sl = pl.ds(sid * RB, RB)
        pltpu.sync_copy(acc.at[sl], part_hbm.at[cid, sl])


@functools.cache
def _make_prop(W):
    return pl.kernel(
        functools.partial(_prop_body, W),
        out_type=jax.ShapeDtypeStruct((NC, N, W), jnp.float32),
        mesh=_mesh(),
        scratch_types=[
            pltpu.VMEM((NCH, CH), jnp.int32),
            pltpu.VMEM((NCH, CH), jnp.int32),
            pltpu.VMEM((2, CH, W), jnp.float32),
            pltpu.VMEM_SHARED((N, W), jnp.float32),
            pltpu.SemaphoreType.DMA,
            pltpu.SemaphoreType.DMA,
            pltpu.SemaphoreType.DMA,
            pltpu.SemaphoreType.DMA,
        ],
        compiler_params=pltpu.CompilerParams(use_tc_tiling_on_sc=False),
    )


def _elu(v):
    return jnp.where(v > 0, v, jnp.exp(jnp.minimum(v, 0.0)) - 1.0)


def _tcb_body(do_ref, di_ref, x_ref, w_ref, t_ref, a_ref, b_ref):
    a1 = lax.rsqrt(jnp.maximum(do_ref[0, 0] + do_ref[0, 1], 1.0))
    b1 = lax.rsqrt(jnp.maximum(di_ref[0, 0] + di_ref[0, 1], 1.0))
    xw = jnp.dot(x_ref[...], w_ref[...], preferred_element_type=jnp.float32)
    t_ref[...] = xw * a1.reshape(RB, 1)
    a_ref[...] = a1.reshape(1, 1, RB)
    b_ref[...] = b1.reshape(1, 1, RB)


def _tcd_body(p_ref, a_ref, b_ref, x_ref, w2_ref, bias_ref, w1p_ref,
              h_ref, t2_ref):
    axw = (p_ref[0] + p_ref[1]) * b_ref[0, 0].reshape(RB, 1)
    xw2 = jnp.dot(x_ref[...], w2_ref[...], preferred_element_type=jnp.float32)
    s = _elu(axw + xw2 + bias_ref[...])
    h = _elu(0.5 * (s[:, :C] + s[:, C:]))
    h_ref[...] = h
    t2_ref[...] = jnp.dot(h, w1p_ref[...],
                          preferred_element_type=jnp.float32) * a_ref[0, 0].reshape(RB, 1)


def _tcf_body(p_ref, b_ref, h_ref, w2p_ref, bp_ref, o_ref):
    z = (p_ref[0] + p_ref[1]) * b_ref[0, 0].reshape(RB, 1)
    logits = z + jnp.dot(h_ref[...], w2p_ref[...],
                         preferred_element_type=jnp.float32) + bp_ref[...]
    m = jnp.max(logits, axis=1, keepdims=True)
    e = jnp.exp(logits - m)
    o_ref[...] = (e / jnp.sum(e, axis=1, keepdims=True))[:, :OUT]


def _row_spec(w):
    return pl.BlockSpec((RB, w), lambda i: (i, 0))


def _part_spec(w):
    return pl.BlockSpec((NC, RB, w), lambda i: (0, i, 0))


def _full_spec(r, w):
    return pl.BlockSpec((r, w), lambda i: (0, 0))


_tcB = pl.pallas_call(
    _tcb_body,
    grid=(N // RB,),
    in_specs=[pl.BlockSpec((1, NC, RB), lambda i: (i, 0, 0)),
              pl.BlockSpec((1, NC, RB), lambda i: (i, 0, 0)),
              _row_spec(F), _full_spec(F, 2 * C)],
    out_specs=[_row_spec(2 * C),
               pl.BlockSpec((1, 1, RB), lambda i: (i, 0, 0)),
               pl.BlockSpec((1, 1, RB), lambda i: (i, 0, 0))],
    out_shape=[
        jax.ShapeDtypeStruct((N, 2 * C), jnp.float32),
        jax.ShapeDtypeStruct((N // RB, 1, RB), jnp.float32),
        jax.ShapeDtypeStruct((N // RB, 1, RB), jnp.float32),
    ],
)

_tcD = pl.pallas_call(
    _tcd_body,
    grid=(N // RB,),
    in_specs=[_part_spec(2 * C),
              pl.BlockSpec((1, 1, RB), lambda i: (i, 0, 0)),
              pl.BlockSpec((1, 1, RB), lambda i: (i, 0, 0)), _row_spec(F),
              _full_spec(F, 2 * C), _full_spec(1, 2 * C), _full_spec(C, 8)],
    out_specs=[_row_spec(C), _row_spec(8)],
    out_shape=[
        jax.ShapeDtypeStruct((N, C), jnp.float32),
        jax.ShapeDtypeStruct((N, 8), jnp.float32),
    ],
)

_tcF = pl.pallas_call(
    _tcf_body,
    grid=(N // RB,),
    in_specs=[_part_spec(8),
              pl.BlockSpec((1, 1, RB), lambda i: (i, 0, 0)), _row_spec(C),
              _full_spec(C, 8), _full_spec(1, 8)],
    out_specs=_row_spec(OUT),
    out_shape=jax.ShapeDtypeStruct((N, OUT), jnp.float32),
)


def kernel(x, edge_index, L1_k0_W1, L1_k0_W2, L1_k0_b, L1_k1_W1, L1_k1_W2,
           L1_k1_b, L2_W1, L2_W2, L2_b):
    src3 = edge_index[0].reshape(NW, NCH, CH)
    dst3 = edge_index[1].reshape(NW, NCH, CH)
    ones_ch = jnp.ones((CH, 8), jnp.float32)
    zeros_n32 = jnp.zeros((N, 2 * C), jnp.float32)
    zeros_n8 = jnp.zeros((N, 8), jnp.float32)

    dego_p, degi_p = _deg_call()(src3, dst3, ones_ch, zeros_n8)

    wcat = jnp.concatenate([L1_k0_W1, L1_k1_W1], axis=1)
    table1, a_n1, b_n1 = _tcB(dego_p, degi_p, x, wcat)

    part1 = _make_prop(2 * C)(table1, src3, dst3, zeros_n32)

    w2cat = jnp.concatenate([L1_k0_W2, L1_k1_W2], axis=1)
    bias01 = jnp.concatenate([L1_k0_b, L1_k1_b]).reshape(1, 2 * C)
    w1p = jnp.pad(L2_W1, ((0, 0), (0, 1)))
    h, table2 = _tcD(part1, a_n1, b_n1, x, w2cat, bias01, w1p)

    part2 = _make_prop(8)(table2, src3, dst3, zeros_n8)

    w2p = jnp.pad(L2_W2, ((0, 0), (0, 1)))
    bp = jnp.concatenate(
        [L2_b, jnp.full((1,), -1e30, jnp.float32)]).reshape(1, 8)
    return _tcF(part2, b_n1, h, w2p, bp)


# speedup vs baseline: 48.5210x; 1.0005x over previous
import functools

import jax
import jax.numpy as jnp
from jax import lax
from jax.experimental import pallas as pl
from jax.experimental.pallas import tpu as pltpu
from jax.experimental.pallas import tpu_sc as plsc

N, E, F, C, OUT = 10000, 320000, 128, 16, 7
NC, NS = 2, 16
NW = NC * NS
EW = E // NW
CH = 1000
NCH = EW // CH
RB = 1000
ZTILES = N // RB

@functools.cache
def _mesh():
    return plsc.VectorSubcoreMesh(
        core_axis_name="c", subcore_axis_name="s", num_cores=NC,
        num_subcores=NS)


def _deg_body(src3, dst3, ones_hbm, zeros_hbm, dego_p, degi_p,
              srcv, dstv, ones_v, dego_s, degi_s, stage_v, outv, sem):
    cid = lax.axis_index("c")
    sid = lax.axis_index("s")
    wid = sid * NC + cid

    @pl.when(sid < ZTILES)
    def _():
        sl = pl.ds(sid * RB, RB)
        pltpu.sync_copy(zeros_hbm.at[sl], dego_s.at[sl])
        pltpu.sync_copy(zeros_hbm.at[sl], degi_s.at[sl])

    pltpu.sync_copy(src3.at[pl.ds(wid * EW, EW)], srcv)
    pltpu.sync_copy(dst3.at[pl.ds(wid * EW, EW)], dstv)
    pltpu.sync_copy(ones_hbm, ones_v)
    plsc.subcore_barrier()

    def chunk(i, carry):
        pltpu.async_copy(ones_v, dego_s.at[srcv.at[pl.ds(i * CH, CH)]],
                         sem, add=True)
        pltpu.async_copy(ones_v, degi_s.at[dstv.at[pl.ds(i * CH, CH)]],
                         sem, add=True)
        return carry

    lax.fori_loop(0, NCH, chunk, 0)

    def drain(i, carry):
        pltpu.make_async_copy(
            ones_v, dego_s.at[srcv.at[pl.ds(0, CH)]], sem).wait()
        pltpu.make_async_copy(
            ones_v, degi_s.at[dstv.at[pl.ds(0, CH)]], sem).wait()
        return carry

    lax.fori_loop(0, NCH, drain, 0)
    plsc.subcore_barrier()

    @pl.when(sid < ZTILES)
    def _():
        sl = pl.ds(sid * RB, RB)
        for acc_s, out_p in ((dego_s, dego_p), (degi_s, degi_p)):
            pltpu.sync_copy(acc_s.at[sl], stage_v.at[pl.ds(0, RB)])
            def ext(j, carry):
                rows = j * 16 + jnp.arange(16, dtype=jnp.int32)
                cols = jnp.zeros((16,), jnp.int32)
                vals = plsc.load_gather(stage_v, [rows, cols])
                outv[pl.ds(j * 16, 16)] = vals
                return carry
            lax.fori_loop(0, (RB + 15) // 16, ext, 0)
            pltpu.sync_copy(outv.at[pl.ds(0, RB)], out_p.at[sid, cid])


@functools.cache
def _deg_call():
    return pl.kernel(
        _deg_body,
        out_type=(
            jax.ShapeDtypeStruct((N // RB, NC, RB), jnp.float32),
            jax.ShapeDtypeStruct((N // RB, NC, RB), jnp.float32),
        ),
        mesh=_mesh(),
        scratch_types=[
            pltpu.VMEM((EW,), jnp.int32),
            pltpu.VMEM((EW,), jnp.int32),
            pltpu.VMEM((CH, 8), jnp.float32),
            pltpu.VMEM_SHARED((N, 8), jnp.float32),
            pltpu.VMEM_SHARED((N, 8), jnp.float32),
            pltpu.VMEM((RB + 8, 8), jnp.float32),
            pltpu.VMEM((RB + 8,), jnp.float32),
            pltpu.SemaphoreType.DMA,
        ],
        compiler_params=pltpu.CompilerParams(
            use_tc_tiling_on_sc=False, needs_layout_passes=False),
    )


def _prop_body(W, table_hbm, src3, dst3, zeros_hbm, part_hbm,
               srcv, dstv, rows, acc, gsem0, gsem1, ssem0, ssem1):
    cid = lax.axis_index("c")
    sid = lax.axis_index("s")
    wid = sid * NC + cid
    gsems = (gsem0, gsem1)
    ssems = (ssem0, ssem1)

    @pl.when(sid < ZTILES)
    def _():
        sl = pl.ds(sid * RB, RB)
        pltpu.sync_copy(zeros_hbm.at[sl], acc.at[sl])

    pltpu.sync_copy(src3.at[pl.ds(wid * EW, EW)], srcv)
    pltpu.sync_copy(dst3.at[pl.ds(wid * EW, EW)], dstv)
    plsc.subcore_barrier()

    for b in range(2):
        pltpu.async_copy(table_hbm.at[srcv.at[pl.ds(b * CH, CH)]],
                         rows.at[b], gsems[b])

    def step(g, carry):
        for b in range(2):
            i = 2 * g + b
            pltpu.make_async_copy(
                table_hbm.at[srcv.at[pl.ds(i * CH, CH)]], rows.at[b],
                gsems[b]).wait()
            pltpu.async_copy(rows.at[b], acc.at[dstv.at[pl.ds(i * CH, CH)]],
                             ssems[b], add=True)
            nxt = i + 2

            @pl.when(nxt < NCH)
            def _():
                pltpu.make_async_copy(
                    rows.at[b], acc.at[dstv.at[pl.ds(i * CH, CH)]],
                    ssems[b]).wait()
                pltpu.async_copy(
                    table_hbm.at[srcv.at[pl.ds(nxt * CH, CH)]], rows.at[b],
                    gsems[b])
        return carry

    lax.fori_loop(0, NCH // 2, step, 0)
    for b in range(2):
        pltpu.make_async_copy(
            rows.at[b], acc.at[dstv.at[pl.ds((NCH - 2 + b) * CH, CH)]],
            ssems[b]).wait()

    plsc.subcore_barrier()

    @pl.when(sid < ZTILES)
    def _():
        sl = pl.ds(sid * RB, RB)
        pltpu.sync_copy(acc.at[sl], part_hbm.at[cid, sl])


@functools.cache
def _make_prop(W):
    return pl.kernel(
        functools.partial(_prop_body, W),
        out_type=jax.ShapeDtypeStruct((NC, N, W), jnp.float32),
        mesh=_mesh(),
        scratch_types=[
            pltpu.VMEM((EW,), jnp.int32),
            pltpu.VMEM((EW,), jnp.int32),
            pltpu.VMEM((2, CH, W), jnp.float32),
            pltpu.VMEM_SHARED((N, W), jnp.float32),
            pltpu.SemaphoreType.DMA,
            pltpu.SemaphoreType.DMA,
            pltpu.SemaphoreType.DMA,
            pltpu.SemaphoreType.DMA,
        ],
        compiler_params=pltpu.CompilerParams(use_tc_tiling_on_sc=False),
    )


def _elu(v):
    return jnp.where(v > 0, v, jnp.exp(jnp.minimum(v, 0.0)) - 1.0)


def _tcb_body(do_ref, di_ref, x_ref, w_ref, t_ref, a_ref, b_ref):
    a1 = lax.rsqrt(jnp.maximum(do_ref[0, 0] + do_ref[0, 1], 1.0))
    b1 = lax.rsqrt(jnp.maximum(di_ref[0, 0] + di_ref[0, 1], 1.0))
    xw = jnp.dot(x_ref[...], w_ref[...], preferred_element_type=jnp.float32)
    t_ref[...] = xw * a1.reshape(RB, 1)
    a_ref[...] = a1.reshape(1, 1, RB)
    b_ref[...] = b1.reshape(1, 1, RB)


def _tcd_body(p_ref, a_ref, b_ref, x_ref, w2_ref, bias_ref, w1p_ref,
              h_ref, t2_ref):
    axw = (p_ref[0] + p_ref[1]) * b_ref[0, 0].reshape(RB, 1)
    xw2 = jnp.dot(x_ref[...], w2_ref[...], preferred_element_type=jnp.float32)
    s = _elu(axw + xw2 + bias_ref[...])
    h = _elu(0.5 * (s[:, :C] + s[:, C:]))
    h_ref[...] = h
    t2_ref[...] = jnp.dot(h, w1p_ref[...],
                          preferred_element_type=jnp.float32) * a_ref[0, 0].reshape(RB, 1)


def _tcf_body(p_ref, b_ref, h_ref, w2p_ref, bp_ref, o_ref):
    z = (p_ref[0] + p_ref[1]) * b_ref[0, 0].reshape(RB, 1)
    logits = z + jnp.dot(h_ref[...], w2p_ref[...],
                         preferred_element_type=jnp.float32) + bp_ref[...]
    m = jnp.max(logits, axis=1, keepdims=True)
    e = jnp.exp(logits - m)
    o_ref[...] = (e / jnp.sum(e, axis=1, keepdims=True))[:, :OUT]


def _row_spec(w):
    return pl.BlockSpec((RB, w), lambda i: (i, 0))


def _part_spec(w):
    return pl.BlockSpec((NC, RB, w), lambda i: (0, i, 0))


def _full_spec(r, w):
    return pl.BlockSpec((r, w), lambda i: (0, 0))


_tcB = pl.pallas_call(
    _tcb_body,
    grid=(N // RB,),
    in_specs=[pl.BlockSpec((1, NC, RB), lambda i: (i, 0, 0)),
              pl.BlockSpec((1, NC, RB), lambda i: (i, 0, 0)),
              _row_spec(F), _full_spec(F, 2 * C)],
    out_specs=[_row_spec(2 * C),
               pl.BlockSpec((1, 1, RB), lambda i: (i, 0, 0)),
               pl.BlockSpec((1, 1, RB), lambda i: (i, 0, 0))],
    out_shape=[
        jax.ShapeDtypeStruct((N, 2 * C), jnp.float32),
        jax.ShapeDtypeStruct((N // RB, 1, RB), jnp.float32),
        jax.ShapeDtypeStruct((N // RB, 1, RB), jnp.float32),
    ],
)

_tcD = pl.pallas_call(
    _tcd_body,
    grid=(N // RB,),
    in_specs=[_part_spec(2 * C),
              pl.BlockSpec((1, 1, RB), lambda i: (i, 0, 0)),
              pl.BlockSpec((1, 1, RB), lambda i: (i, 0, 0)), _row_spec(F),
              _full_spec(F, 2 * C), _full_spec(1, 2 * C), _full_spec(C, 8)],
    out_specs=[_row_spec(C), _row_spec(8)],
    out_shape=[
        jax.ShapeDtypeStruct((N, C), jnp.float32),
        jax.ShapeDtypeStruct((N, 8), jnp.float32),
    ],
)

_tcF = pl.pallas_call(
    _tcf_body,
    grid=(N // RB,),
    in_specs=[_part_spec(8),
              pl.BlockSpec((1, 1, RB), lambda i: (i, 0, 0)), _row_spec(C),
              _full_spec(C, 8), _full_spec(1, 8)],
    out_specs=_row_spec(OUT),
    out_shape=jax.ShapeDtypeStruct((N, OUT), jnp.float32),
)


def kernel(x, edge_index, L1_k0_W1, L1_k0_W2, L1_k0_b, L1_k1_W1, L1_k1_W2,
           L1_k1_b, L2_W1, L2_W2, L2_b):
    src3 = edge_index[0]
    dst3 = edge_index[1]
    ones_ch = jnp.ones((CH, 8), jnp.float32)
    zeros_n32 = jnp.zeros((N, 2 * C), jnp.float32)
    zeros_n8 = jnp.zeros((N, 8), jnp.float32)

    dego_p, degi_p = _deg_call()(src3, dst3, ones_ch, zeros_n8)

    wcat = jnp.concatenate([L1_k0_W1, L1_k1_W1], axis=1)
    table1, a_n1, b_n1 = _tcB(dego_p, degi_p, x, wcat)

    part1 = _make_prop(2 * C)(table1, src3, dst3, zeros_n32)

    w2cat = jnp.concatenate([L1_k0_W2, L1_k1_W2], axis=1)
    bias01 = jnp.concatenate([L1_k0_b, L1_k1_b]).reshape(1, 2 * C)
    w1p = jnp.pad(L2_W1, ((0, 0), (0, 1)))
    h, table2 = _tcD(part1, a_n1, b_n1, x, w2cat, bias01, w1p)

    part2 = _make_prop(8)(table2, src3, dst3, zeros_n8)

    w2p = jnp.pad(L2_W2, ((0, 0), (0, 1)))
    bp = jnp.concatenate(
        [L2_b, jnp.full((1,), -1e30, jnp.float32)]).reshape(1, 8)
    return _tcF(part2, b_n1, h, w2p, bp)
